# baseline (device time: 575753 ns/iter reference)
import jax
import jax.numpy as jnp
from jax import lax
from jax.experimental import pallas as pl
from jax.experimental.pallas import tpu as pltpu

N_DEV = 4
M_PER = 2048
K = 8192
N_SH = 1024
TILE = 512
HALF_M = 1024
WHALF = K // 2
WQ = K // 4
XC = 256


def _gelu(y):
    c = 0.7978845608028654
    return 0.5 * y * (1.0 + jnp.tanh(c * (y + 0.044715 * y * y * y)))


def kernel(x, w_mat):
    def body(x_ref, w_ref, out_ref, wg_ref, xbf_ref, res_ref,
             xtile, otile, cf32, wf32, wbf, w_vmem,
             send_a, recv_a, send_b, recv_b, send_res, recv_res,
             in_sem, out_sem, stage_sem):
        me = lax.axis_index("i")
        left = lax.rem(me + 3, N_DEV)
        right = lax.rem(me + 1, N_DEV)
        diag = lax.rem(me + 2, N_DEV)

        barrier = pltpu.get_barrier_semaphore()
        for nbr in (left, right):
            pl.semaphore_signal(barrier, inc=1, device_id=(nbr,),
                                device_id_type=pl.DeviceIdType.MESH)
        pl.semaphore_wait(barrier, 2)

        def cast_w_quarter(q):
            cp = pltpu.make_async_copy(
                w_ref.at[pl.ds(q * WQ, WQ), :], wf32, in_sem)
            cp.start()
            cp.wait()
            wbf[...] = wf32[...].astype(jnp.bfloat16)
            w_vmem[pl.ds(q * WQ, WQ), :] = wbf[...]
            cp2 = pltpu.make_async_copy(
                wbf, wg_ref.at[me, pl.ds(q * WQ, WQ), :], out_sem)
            cp2.start()
            cp2.wait()

        def ring_rdma(origin, row0, sems_idx, to_right):
            sl = (origin, pl.ds(row0, WHALF))
            return pltpu.make_async_remote_copy(
                src_ref=wg_ref.at[sl], dst_ref=wg_ref.at[sl],
                send_sem=(send_a if to_right else send_b).at[sems_idx],
                recv_sem=(recv_a if to_right else recv_b).at[sems_idx],
                device_id=(right if to_right else left,),
                device_id_type=pl.DeviceIdType.MESH)

        def own_tiles(lo, hi):
            def tile_body(t, carry):
                cp_in = pltpu.make_async_copy(
                    xbf_ref.at[pl.ds(t * TILE, TILE), :], xtile, in_sem)
                cp_in.start()
                cp_in.wait()
                y = jnp.dot(xtile[...], w_vmem[...],
                            preferred_element_type=jnp.float32)
                otile[...] = _gelu(y)
                cp_out = pltpu.make_async_copy(
                    otile, out_ref.at[pl.ds(me * M_PER + t * TILE, TILE), :],
                    out_sem)
                cp_out.start()
                cp_out.wait()
                return carry
            lax.fori_loop(lo, hi, tile_body, 0)

        def stage_w(peer):
            cp_w = pltpu.make_async_copy(wg_ref.at[peer], w_vmem, stage_sem)
            cp_w.start()
            cp_w.wait()

        def block_tiles(slot, lo, hi):
            def tile_body(t, carry):
                cp_in = pltpu.make_async_copy(
                    xbf_ref.at[pl.ds(t * TILE, TILE), :], xtile, in_sem)
                cp_in.start()
                cp_in.wait()
                y = jnp.dot(xtile[...], w_vmem[...],
                            preferred_element_type=jnp.float32)
                otile[...] = _gelu(y)
                cp_out = pltpu.make_async_copy(
                    otile, res_ref.at[slot, pl.ds(t * TILE, TILE), :],
                    out_sem)
                cp_out.start()
                cp_out.wait()
                return carry
            lax.fori_loop(lo, hi, tile_body, 0)

        def res_rdma(slot, row0, rows, sem_idx, peer):
            return pltpu.make_async_remote_copy(
                src_ref=res_ref.at[slot, pl.ds(row0, rows), :],
                dst_ref=out_ref.at[pl.ds(me * M_PER + row0, rows), :],
                send_sem=send_res.at[sem_idx], recv_sem=recv_res.at[sem_idx],
                device_id=(peer,), device_id_type=pl.DeviceIdType.MESH)

        cast_w_quarter(0)
        cast_w_quarter(1)
        a1 = ring_rdma(me, 0, 0, True)
        a1.start()
        cast_w_quarter(2)
        cast_w_quarter(3)
        b1 = ring_rdma(me, WHALF, 0, False)
        b1.start()

        for i in range(M_PER // XC):
            cp = pltpu.make_async_copy(
                x_ref.at[pl.ds(i * XC, XC), :], cf32, in_sem)
            cp.start()
            cp.wait()
            xtile[pl.ds(0, XC), :] = cf32[...].astype(jnp.bfloat16)
            cp2 = pltpu.make_async_copy(
                xtile.at[pl.ds(0, XC), :], xbf_ref.at[pl.ds(i * XC, XC), :],
                out_sem)
            cp2.start()
            cp2.wait()

        own_tiles(0, 4)
        a1.wait()
        b1.wait()

        a2 = ring_rdma(lax.rem(me + 3, N_DEV), 0, 1, True)
        b2 = ring_rdma(lax.rem(me + 1, N_DEV), WHALF, 1, False)
        a2.start()
        b2.start()
        a2.wait()
        b2.wait()

        a3 = ring_rdma(diag, 0, 2, True)
        b3 = ring_rdma(diag, WHALF, 2, False)
        a3.start()
        b3.start()
        stage_w(diag)
        block_tiles(1, 0, 4)
        r_d = res_rdma(1, 0, M_PER, 2, diag)
        r_d.start()
        a3.wait()
        b3.wait()

        stage_w(right)
        block_tiles(0, 0, 2)
        r_r0 = res_rdma(0, 0, HALF_M, 0, right)
        r_r0.start()
        block_tiles(0, 2, 4)
        r_r1 = res_rdma(0, HALF_M, HALF_M, 1, right)
        r_r1.start()
        stage_w(left)
        block_tiles(2, 0, 2)
        r_l0 = res_rdma(2, 0, HALF_M, 3, left)
        r_l0.start()
        block_tiles(2, 2, 4)
        r_l1 = res_rdma(2, HALF_M, HALF_M, 4, left)
        r_l1.start()

        for d in (r_r0, r_r1, r_d, r_l0, r_l1):
            d.wait_send()
        for s, row0, rows, idx in (
                (left, 0, HALF_M, 0), (left, HALF_M, HALF_M, 1),
                (diag, 0, M_PER, 2),
                (right, 0, HALF_M, 3), (right, HALF_M, HALF_M, 4)):
            d = pltpu.make_async_remote_copy(
                src_ref=res_ref.at[0, pl.ds(row0, rows), :],
                dst_ref=out_ref.at[pl.ds(s * M_PER + row0, rows), :],
                send_sem=send_res.at[idx], recv_sem=recv_res.at[idx],
                device_id=(me,), device_id_type=pl.DeviceIdType.MESH)
            d.wait_recv()

    out = pl.pallas_call(
        body,
        out_shape=[
            jax.ShapeDtypeStruct((N_DEV * M_PER, N_SH), jnp.float32),
            jax.ShapeDtypeStruct((N_DEV, K, N_SH), jnp.bfloat16),
            jax.ShapeDtypeStruct((M_PER, K), jnp.bfloat16),
            jax.ShapeDtypeStruct((3, M_PER, N_SH), jnp.float32),
        ],
        in_specs=[
            pl.BlockSpec(memory_space=pl.ANY),
            pl.BlockSpec(memory_space=pl.ANY),
        ],
        out_specs=[pl.BlockSpec(memory_space=pl.ANY)] * 4,
        scratch_shapes=[
            pltpu.VMEM((TILE, K), jnp.bfloat16),
            pltpu.VMEM((TILE, N_SH), jnp.float32),
            pltpu.VMEM((XC, K), jnp.float32),
            pltpu.VMEM((WQ, N_SH), jnp.float32),
            pltpu.VMEM((WQ, N_SH), jnp.bfloat16),
            pltpu.VMEM((K, N_SH), jnp.bfloat16),
            pltpu.SemaphoreType.DMA((3,)),
            pltpu.SemaphoreType.DMA((3,)),
            pltpu.SemaphoreType.DMA((3,)),
            pltpu.SemaphoreType.DMA((3,)),
            pltpu.SemaphoreType.DMA((5,)),
            pltpu.SemaphoreType.DMA((5,)),
            pltpu.SemaphoreType.DMA,
            pltpu.SemaphoreType.DMA,
            pltpu.SemaphoreType.DMA,
        ],
        compiler_params=pltpu.CompilerParams(
            collective_id=0, vmem_limit_bytes=60 * 1024 * 1024),
    )(x, w_mat)
    return out[0]
